# baseline (speedup 1.0000x reference)
import functools

import jax
import jax.numpy as jnp
from jax.experimental import pallas as pl
from jax.experimental.pallas import tpu as pltpu

_B, _S, _H = 4, 4096, 2048
_E, _K = 16, 2
_T = _B * _S
_TB = 512
_GRID = _T // _TB
_MIN_PROB = 0.001


def _router_kernel(h_ref, wt_ref, noise_ref,
                   d0_ref, d1_ref, c0_ref, c1_ref,
                   un_ref, ll_ref, il_ref, zl_ref, el_ref, ee_ref,
                   acc_scores, acc_usage, acc_ent, acc_z):
    i = pl.program_id(0)

    @pl.when(i == 0)
    def _init():
        acc_scores[...] = jnp.zeros_like(acc_scores)
        acc_usage[...] = jnp.zeros_like(acc_usage)
        acc_ent[...] = jnp.zeros_like(acc_ent)
        acc_z[...] = jnp.zeros_like(acc_z)

    raw = jnp.dot(h_ref[...], wt_ref[...], preferred_element_type=jnp.float32)
    logits = raw + noise_ref[...]

    m = jnp.max(logits, axis=-1, keepdims=True)
    ex = jnp.exp(logits - m)
    sm = ex / jnp.sum(ex, axis=-1, keepdims=True)

    scores = sm * (1.0 - _MIN_PROB * _E) + _MIN_PROB
    scores = scores / jnp.sum(scores, axis=-1, keepdims=True)

    e_iota = jax.lax.broadcasted_iota(jnp.int32, scores.shape, 1)
    m1 = jnp.max(scores, axis=-1, keepdims=True)
    i1 = jnp.min(jnp.where(scores == m1, e_iota, _E), axis=-1, keepdims=True)
    masked = jnp.where(e_iota == i1, -jnp.inf, scores)
    m2 = jnp.max(masked, axis=-1, keepdims=True)
    i2 = jnp.min(jnp.where(masked == m2, e_iota, _E), axis=-1, keepdims=True)

    denom = m1 + m2
    d0 = (e_iota == i1).astype(jnp.float32)
    d1 = (e_iota == i2).astype(jnp.float32)
    d0_ref[...] = d0
    d1_ref[...] = d1
    c0_ref[...] = d0 * (m1 / denom)
    c1_ref[...] = d1 * (m2 / denom)

    acc_scores[...] += jnp.sum(scores, axis=0, keepdims=True)
    acc_usage[...] += jnp.sum(d0 + d1, axis=0, keepdims=True)
    ent = -jnp.sum(scores * jnp.log(scores + 1e-10), axis=-1, keepdims=True)
    acc_ent[...] += jnp.sum(ent, axis=0, keepdims=True)
    ms = jnp.max(scores, axis=-1, keepdims=True)
    lse = ms + jnp.log(jnp.sum(jnp.exp(scores - ms), axis=-1, keepdims=True))
    acc_z[...] += jnp.sum(lse * lse, axis=0, keepdims=True)

    @pl.when(i == _GRID - 1)
    def _finalize():
        def put(ref, val):
            ref[...] = jnp.broadcast_to(val, (1, 1)).astype(jnp.float32)

        usage = acc_usage[...]
        total = jnp.sum(usage) + 1e-10
        un = usage / total
        un_ref[...] = un
        gate_probs = acc_scores[...] / _T
        put(ll_ref, jnp.clip(jnp.sum(gate_probs * un) * _E, 0.0, 10.0))
        imp_mean = jnp.mean(gate_probs)
        imp_var = jnp.mean((gate_probs - imp_mean) ** 2)
        put(il_ref, jnp.clip(imp_var / (imp_mean * imp_mean + 1e-10), 0.0, 1.0))
        ent_mean = acc_ent[...] / _T
        put(el_ref, jnp.clip((1.0 - ent_mean / jnp.log(jnp.float32(_E))) * 5.0,
                             0.0, 5.0))
        put(zl_ref, jnp.clip(acc_z[...] / _T, 0.0, 100.0))
        put(ee_ref, -jnp.sum(un * jnp.log(un + 1e-10)))


@functools.partial(jax.jit, static_argnums=())
def _run(flat, gate_Wt, noise):
    fo = jax.ShapeDtypeStruct
    outs = pl.pallas_call(
        _router_kernel,
        grid=(_GRID,),
        in_specs=[
            pl.BlockSpec((_TB, _H), lambda i: (i, 0)),
            pl.BlockSpec((_H, _E), lambda i: (0, 0)),
            pl.BlockSpec((_TB, _E), lambda i: (i, 0)),
        ],
        out_specs=[
            pl.BlockSpec((_TB, _E), lambda i: (i, 0)),
            pl.BlockSpec((_TB, _E), lambda i: (i, 0)),
            pl.BlockSpec((_TB, _E), lambda i: (i, 0)),
            pl.BlockSpec((_TB, _E), lambda i: (i, 0)),
            pl.BlockSpec((1, _E), lambda i: (0, 0)),
            pl.BlockSpec((1, 1), lambda i: (0, 0)),
            pl.BlockSpec((1, 1), lambda i: (0, 0)),
            pl.BlockSpec((1, 1), lambda i: (0, 0)),
            pl.BlockSpec((1, 1), lambda i: (0, 0)),
            pl.BlockSpec((1, 1), lambda i: (0, 0)),
        ],
        out_shape=[
            fo((_T, _E), jnp.float32),
            fo((_T, _E), jnp.float32),
            fo((_T, _E), jnp.float32),
            fo((_T, _E), jnp.float32),
            fo((1, _E), jnp.float32),
            fo((1, 1), jnp.float32),
            fo((1, 1), jnp.float32),
            fo((1, 1), jnp.float32),
            fo((1, 1), jnp.float32),
            fo((1, 1), jnp.float32),
        ],
        scratch_shapes=[
            pltpu.VMEM((1, _E), jnp.float32),
            pltpu.VMEM((1, _E), jnp.float32),
            pltpu.VMEM((1, 1), jnp.float32),
            pltpu.VMEM((1, 1), jnp.float32),
        ],
    )(flat, gate_Wt, noise)
    return outs


def kernel(hidden_states, gate_W):
    b, s, h = hidden_states.shape
    flat = hidden_states.reshape(b * s, h).astype(jnp.float32)

    u = jax.random.uniform(jax.random.key(1), (b * s, _E), dtype=jnp.float32)
    noise = -jnp.log(-jnp.log(u + 1e-10) + 1e-10) * 0.1

    (d0, d1, c0, c1, un, ll, il, zl, el, ee) = _run(flat, gate_W.T, noise)

    dispatch = jnp.stack([d0, d1], axis=-1).astype(bool).reshape(b, s, _E, _K)
    combine = jnp.stack([c0, c1], axis=-1).reshape(b, s, _E, _K)
    return (dispatch, combine,
            ll[0, 0], il[0, 0], zl[0, 0], el[0, 0],
            un[0], ee[0, 0])


# speedup vs baseline: 1.1014x; 1.1014x over previous
import functools

import jax
import jax.numpy as jnp
from jax.experimental import pallas as pl
from jax.experimental.pallas import tpu as pltpu

_B, _S, _H = 4, 4096, 2048
_E, _K = 16, 2
_T = _B * _S
_TB = 1024
_GRID = _T // _TB
_MIN_PROB = 0.001


def _router_kernel(h_ref, h2_ref, wt_ref, noise_ref,
                   d0_ref, d1_ref, c0_ref, c1_ref,
                   un_ref, ll_ref, il_ref, zl_ref, el_ref, ee_ref,
                   acc_scores, acc_usage, acc_ent, acc_z):
    i = pl.program_id(0)

    @pl.when(i == 0)
    def _init():
        acc_scores[...] = jnp.zeros_like(acc_scores)
        acc_usage[...] = jnp.zeros_like(acc_usage)
        acc_ent[...] = jnp.zeros_like(acc_ent)
        acc_z[...] = jnp.zeros_like(acc_z)

    raw = (jnp.dot(h_ref[...], wt_ref[:_H // 2], preferred_element_type=jnp.float32)
           + jnp.dot(h2_ref[...], wt_ref[_H // 2:], preferred_element_type=jnp.float32))
    logits = raw + noise_ref[...]

    m = jnp.max(logits, axis=-1, keepdims=True)
    ex = jnp.exp(logits - m)
    sm = ex / jnp.sum(ex, axis=-1, keepdims=True)

    scores = sm * (1.0 - _MIN_PROB * _E) + _MIN_PROB
    scores = scores / jnp.sum(scores, axis=-1, keepdims=True)

    e_iota = jax.lax.broadcasted_iota(jnp.int32, scores.shape, 1)
    m1 = jnp.max(scores, axis=-1, keepdims=True)
    i1 = jnp.min(jnp.where(scores == m1, e_iota, _E), axis=-1, keepdims=True)
    masked = jnp.where(e_iota == i1, -jnp.inf, scores)
    m2 = jnp.max(masked, axis=-1, keepdims=True)
    i2 = jnp.min(jnp.where(masked == m2, e_iota, _E), axis=-1, keepdims=True)

    denom = m1 + m2
    d0 = (e_iota == i1).astype(jnp.float32)
    d1 = (e_iota == i2).astype(jnp.float32)
    d0_ref[...] = d0
    d1_ref[...] = d1
    c0_ref[...] = d0 * (m1 / denom)
    c1_ref[...] = d1 * (m2 / denom)

    acc_scores[...] += jnp.sum(scores, axis=0, keepdims=True)
    acc_usage[...] += jnp.sum(d0 + d1, axis=0, keepdims=True)
    ent = -jnp.sum(scores * jnp.log(scores + 1e-10), axis=-1, keepdims=True)
    acc_ent[...] += jnp.sum(ent, axis=0, keepdims=True)
    ms = jnp.max(scores, axis=-1, keepdims=True)
    lse = ms + jnp.log(jnp.sum(jnp.exp(scores - ms), axis=-1, keepdims=True))
    acc_z[...] += jnp.sum(lse * lse, axis=0, keepdims=True)

    @pl.when(i == _GRID - 1)
    def _finalize():
        def put(ref, val):
            ref[...] = jnp.broadcast_to(val, (1, 1)).astype(jnp.float32)

        usage = acc_usage[...]
        total = jnp.sum(usage) + 1e-10
        un = usage / total
        un_ref[...] = un
        gate_probs = acc_scores[...] / _T
        put(ll_ref, jnp.clip(jnp.sum(gate_probs * un) * _E, 0.0, 10.0))
        imp_mean = jnp.mean(gate_probs)
        imp_var = jnp.mean((gate_probs - imp_mean) ** 2)
        put(il_ref, jnp.clip(imp_var / (imp_mean * imp_mean + 1e-10), 0.0, 1.0))
        ent_mean = acc_ent[...] / _T
        put(el_ref, jnp.clip((1.0 - ent_mean / jnp.log(jnp.float32(_E))) * 5.0,
                             0.0, 5.0))
        put(zl_ref, jnp.clip(acc_z[...] / _T, 0.0, 100.0))
        put(ee_ref, -jnp.sum(un * jnp.log(un + 1e-10)))


@functools.partial(jax.jit, static_argnums=())
def _run(flat, gate_Wt, noise):
    fo = jax.ShapeDtypeStruct
    outs = pl.pallas_call(
        _router_kernel,
        grid=(_GRID,),
        in_specs=[
            pl.BlockSpec((_TB, _H // 2), lambda i: (i, 0)),
            pl.BlockSpec((_TB, _H // 2), lambda i: (i, 1)),
            pl.BlockSpec((_H, _E), lambda i: (0, 0)),
            pl.BlockSpec((_TB, _E), lambda i: (i, 0)),
        ],
        out_specs=[
            pl.BlockSpec((_TB, _E), lambda i: (i, 0)),
            pl.BlockSpec((_TB, _E), lambda i: (i, 0)),
            pl.BlockSpec((_TB, _E), lambda i: (i, 0)),
            pl.BlockSpec((_TB, _E), lambda i: (i, 0)),
            pl.BlockSpec((1, _E), lambda i: (0, 0)),
            pl.BlockSpec((1, 1), lambda i: (0, 0)),
            pl.BlockSpec((1, 1), lambda i: (0, 0)),
            pl.BlockSpec((1, 1), lambda i: (0, 0)),
            pl.BlockSpec((1, 1), lambda i: (0, 0)),
            pl.BlockSpec((1, 1), lambda i: (0, 0)),
        ],
        out_shape=[
            fo((_T, _E), jnp.float32),
            fo((_T, _E), jnp.float32),
            fo((_T, _E), jnp.float32),
            fo((_T, _E), jnp.float32),
            fo((1, _E), jnp.float32),
            fo((1, 1), jnp.float32),
            fo((1, 1), jnp.float32),
            fo((1, 1), jnp.float32),
            fo((1, 1), jnp.float32),
            fo((1, 1), jnp.float32),
        ],
        scratch_shapes=[
            pltpu.VMEM((1, _E), jnp.float32),
            pltpu.VMEM((1, _E), jnp.float32),
            pltpu.VMEM((1, 1), jnp.float32),
            pltpu.VMEM((1, 1), jnp.float32),
        ],
    )(flat, flat, gate_Wt, noise)
    return outs


def kernel(hidden_states, gate_W):
    b, s, h = hidden_states.shape
    flat = hidden_states.reshape(b * s, h).astype(jnp.float32)

    u = jax.random.uniform(jax.random.key(1), (b * s, _E), dtype=jnp.float32)
    noise = -jnp.log(-jnp.log(u + 1e-10) + 1e-10) * 0.1

    (d0, d1, c0, c1, un, ll, il, zl, el, ee) = _run(flat, gate_W.T, noise)

    dispatch = jnp.stack([d0, d1], axis=-1).astype(bool).reshape(b, s, _E, _K)
    combine = jnp.stack([c0, c1], axis=-1).reshape(b, s, _E, _K)
    return (dispatch, combine,
            ll[0, 0], il[0, 0], zl[0, 0], el[0, 0],
            un[0], ee[0, 0])


# speedup vs baseline: 1.1034x; 1.0018x over previous
import functools

import jax
import jax.numpy as jnp
from jax.experimental import pallas as pl
from jax.experimental.pallas import tpu as pltpu

_B, _S, _H = 4, 4096, 2048
_E, _K = 16, 2
_T = _B * _S
_TB = 1024
_GRID = _T // _TB
_MIN_PROB = 0.001


def _router_kernel(h_ref, h2_ref, h3_ref, h4_ref, wt_ref, noise_ref,
                   d0_ref, d1_ref, c0_ref, c1_ref,
                   un_ref, ll_ref, il_ref, zl_ref, el_ref, ee_ref,
                   acc_scores, acc_usage, acc_ent, acc_z):
    i = pl.program_id(0)

    @pl.when(i == 0)
    def _init():
        acc_scores[...] = jnp.zeros_like(acc_scores)
        acc_usage[...] = jnp.zeros_like(acc_usage)
        acc_ent[...] = jnp.zeros_like(acc_ent)
        acc_z[...] = jnp.zeros_like(acc_z)

    q = _H // 4
    raw = (jnp.dot(h_ref[...], wt_ref[:q], preferred_element_type=jnp.float32)
           + jnp.dot(h2_ref[...], wt_ref[q:2 * q], preferred_element_type=jnp.float32)
           + jnp.dot(h3_ref[...], wt_ref[2 * q:3 * q], preferred_element_type=jnp.float32)
           + jnp.dot(h4_ref[...], wt_ref[3 * q:], preferred_element_type=jnp.float32))
    logits = raw + noise_ref[...]

    m = jnp.max(logits, axis=-1, keepdims=True)
    ex = jnp.exp(logits - m)
    sm = ex / jnp.sum(ex, axis=-1, keepdims=True)

    scores = sm * (1.0 - _MIN_PROB * _E) + _MIN_PROB
    scores = scores / jnp.sum(scores, axis=-1, keepdims=True)

    e_iota = jax.lax.broadcasted_iota(jnp.int32, scores.shape, 1)
    m1 = jnp.max(scores, axis=-1, keepdims=True)
    i1 = jnp.min(jnp.where(scores == m1, e_iota, _E), axis=-1, keepdims=True)
    masked = jnp.where(e_iota == i1, -jnp.inf, scores)
    m2 = jnp.max(masked, axis=-1, keepdims=True)
    i2 = jnp.min(jnp.where(masked == m2, e_iota, _E), axis=-1, keepdims=True)

    denom = m1 + m2
    d0 = (e_iota == i1).astype(jnp.float32)
    d1 = (e_iota == i2).astype(jnp.float32)
    d0_ref[...] = d0
    d1_ref[...] = d1
    c0_ref[...] = d0 * (m1 / denom)
    c1_ref[...] = d1 * (m2 / denom)

    acc_scores[...] += jnp.sum(scores, axis=0, keepdims=True)
    acc_usage[...] += jnp.sum(d0 + d1, axis=0, keepdims=True)
    ent = -jnp.sum(scores * jnp.log(scores + 1e-10), axis=-1, keepdims=True)
    acc_ent[...] += jnp.sum(ent, axis=0, keepdims=True)
    ms = jnp.max(scores, axis=-1, keepdims=True)
    lse = ms + jnp.log(jnp.sum(jnp.exp(scores - ms), axis=-1, keepdims=True))
    acc_z[...] += jnp.sum(lse * lse, axis=0, keepdims=True)

    @pl.when(i == _GRID - 1)
    def _finalize():
        def put(ref, val):
            ref[...] = jnp.broadcast_to(val, (1, 1)).astype(jnp.float32)

        usage = acc_usage[...]
        total = jnp.sum(usage) + 1e-10
        un = usage / total
        un_ref[...] = un
        gate_probs = acc_scores[...] / _T
        put(ll_ref, jnp.clip(jnp.sum(gate_probs * un) * _E, 0.0, 10.0))
        imp_mean = jnp.mean(gate_probs)
        imp_var = jnp.mean((gate_probs - imp_mean) ** 2)
        put(il_ref, jnp.clip(imp_var / (imp_mean * imp_mean + 1e-10), 0.0, 1.0))
        ent_mean = acc_ent[...] / _T
        put(el_ref, jnp.clip((1.0 - ent_mean / jnp.log(jnp.float32(_E))) * 5.0,
                             0.0, 5.0))
        put(zl_ref, jnp.clip(acc_z[...] / _T, 0.0, 100.0))
        put(ee_ref, -jnp.sum(un * jnp.log(un + 1e-10)))


@functools.partial(jax.jit, static_argnums=())
def _run(flat, gate_Wt, noise):
    fo = jax.ShapeDtypeStruct
    outs = pl.pallas_call(
        _router_kernel,
        grid=(_GRID,),
        in_specs=[
            pl.BlockSpec((_TB, _H // 4), lambda i: (i, 0)),
            pl.BlockSpec((_TB, _H // 4), lambda i: (i, 1)),
            pl.BlockSpec((_TB, _H // 4), lambda i: (i, 2)),
            pl.BlockSpec((_TB, _H // 4), lambda i: (i, 3)),
            pl.BlockSpec((_H, _E), lambda i: (0, 0)),
            pl.BlockSpec((_TB, _E), lambda i: (i, 0)),
        ],
        out_specs=[
            pl.BlockSpec((_TB, _E), lambda i: (i, 0)),
            pl.BlockSpec((_TB, _E), lambda i: (i, 0)),
            pl.BlockSpec((_TB, _E), lambda i: (i, 0)),
            pl.BlockSpec((_TB, _E), lambda i: (i, 0)),
            pl.BlockSpec((1, _E), lambda i: (0, 0)),
            pl.BlockSpec((1, 1), lambda i: (0, 0)),
            pl.BlockSpec((1, 1), lambda i: (0, 0)),
            pl.BlockSpec((1, 1), lambda i: (0, 0)),
            pl.BlockSpec((1, 1), lambda i: (0, 0)),
            pl.BlockSpec((1, 1), lambda i: (0, 0)),
        ],
        out_shape=[
            fo((_T, _E), jnp.float32),
            fo((_T, _E), jnp.float32),
            fo((_T, _E), jnp.float32),
            fo((_T, _E), jnp.float32),
            fo((1, _E), jnp.float32),
            fo((1, 1), jnp.float32),
            fo((1, 1), jnp.float32),
            fo((1, 1), jnp.float32),
            fo((1, 1), jnp.float32),
            fo((1, 1), jnp.float32),
        ],
        scratch_shapes=[
            pltpu.VMEM((1, _E), jnp.float32),
            pltpu.VMEM((1, _E), jnp.float32),
            pltpu.VMEM((1, 1), jnp.float32),
            pltpu.VMEM((1, 1), jnp.float32),
        ],
    )(flat, flat, flat, flat, gate_Wt, noise)
    return outs


def kernel(hidden_states, gate_W):
    b, s, h = hidden_states.shape
    flat = hidden_states.reshape(b * s, h).astype(jnp.float32)

    u = jax.random.uniform(jax.random.key(1), (b * s, _E), dtype=jnp.float32)
    noise = -jnp.log(-jnp.log(u + 1e-10) + 1e-10) * 0.1

    (d0, d1, c0, c1, un, ll, il, zl, el, ee) = _run(flat, gate_W.T, noise)

    dispatch = jnp.stack([d0, d1], axis=-1).astype(bool).reshape(b, s, _E, _K)
    combine = jnp.stack([c0, c1], axis=-1).reshape(b, s, _E, _K)
    return (dispatch, combine,
            ll[0, 0], il[0, 0], zl[0, 0], el[0, 0],
            un[0], ee[0, 0])


# speedup vs baseline: 1.2152x; 1.1013x over previous
import jax
import jax.numpy as jnp
from jax.experimental import pallas as pl
from jax.experimental.pallas import tpu as pltpu

_B, _S, _H = 4, 4096, 2048
_E, _K = 16, 2
_T = _B * _S
_TB = 1024
_GRID = _T // _TB
_MIN_PROB = 0.001


def _router_kernel(h1_ref, h2_ref, wt_ref, noise_ref,
                   disp_ref, comb_ref,
                   un_ref, ll_ref, il_ref, zl_ref, el_ref, ee_ref,
                   acc_scores, acc_usage, acc_ent, acc_z):
    i = pl.program_id(0)

    @pl.when(i == 0)
    def _init():
        acc_scores[...] = jnp.zeros_like(acc_scores)
        acc_usage[...] = jnp.zeros_like(acc_usage)
        acc_ent[...] = jnp.zeros_like(acc_ent)
        acc_z[...] = jnp.zeros_like(acc_z)

    half = _H // 2
    raw = (jnp.dot(h1_ref[...], wt_ref[:half], preferred_element_type=jnp.float32)
           + jnp.dot(h2_ref[...], wt_ref[half:], preferred_element_type=jnp.float32))
    logits = raw + noise_ref[...]

    m = jnp.max(logits, axis=-1, keepdims=True)
    ex = jnp.exp(logits - m)
    sm = ex / jnp.sum(ex, axis=-1, keepdims=True)

    scores = sm * (1.0 - _MIN_PROB * _E) + _MIN_PROB
    scores = scores / jnp.sum(scores, axis=-1, keepdims=True)

    e_iota = jax.lax.broadcasted_iota(jnp.int32, scores.shape, 1)
    m1 = jnp.max(scores, axis=-1, keepdims=True)
    i1 = jnp.min(jnp.where(scores == m1, e_iota, _E), axis=-1, keepdims=True)
    masked = jnp.where(e_iota == i1, -jnp.inf, scores)
    m2 = jnp.max(masked, axis=-1, keepdims=True)
    i2 = jnp.min(jnp.where(masked == m2, e_iota, _E), axis=-1, keepdims=True)

    denom = m1 + m2
    w1 = m1 / denom
    w2 = m2 / denom

    lane = jax.lax.broadcasted_iota(jnp.int32, (_TB, _E * _K), 1)
    e_idx = lane // _K
    is0 = (lane - e_idx * _K) == 0
    hit = e_idx == jnp.where(is0, i1, i2)
    disp_ref[...] = hit.astype(jnp.float32)
    comb_ref[...] = jnp.where(hit, jnp.where(is0, w1, w2), 0.0)

    d0 = (e_iota == i1).astype(jnp.float32)
    d1 = (e_iota == i2).astype(jnp.float32)
    acc_scores[...] += jnp.sum(scores, axis=0, keepdims=True)
    acc_usage[...] += jnp.sum(d0 + d1, axis=0, keepdims=True)
    ent = -jnp.sum(scores * jnp.log(scores + 1e-10), axis=-1, keepdims=True)
    acc_ent[...] += jnp.sum(ent, axis=0, keepdims=True)
    ms = jnp.max(scores, axis=-1, keepdims=True)
    lse = ms + jnp.log(jnp.sum(jnp.exp(scores - ms), axis=-1, keepdims=True))
    acc_z[...] += jnp.sum(lse * lse, axis=0, keepdims=True)

    @pl.when(i == _GRID - 1)
    def _finalize():
        def put(ref, val):
            ref[...] = jnp.broadcast_to(val, (1, 1)).astype(jnp.float32)

        usage = acc_usage[...]
        total = jnp.sum(usage) + 1e-10
        un = usage / total
        un_ref[...] = un
        gate_probs = acc_scores[...] / _T
        put(ll_ref, jnp.clip(jnp.sum(gate_probs * un) * _E, 0.0, 10.0))
        imp_mean = jnp.mean(gate_probs)
        imp_var = jnp.mean((gate_probs - imp_mean) ** 2)
        put(il_ref, jnp.clip(imp_var / (imp_mean * imp_mean + 1e-10), 0.0, 1.0))
        ent_mean = acc_ent[...] / _T
        put(el_ref, jnp.clip((1.0 - ent_mean / jnp.log(jnp.float32(_E))) * 5.0,
                             0.0, 5.0))
        put(zl_ref, jnp.clip(acc_z[...] / _T, 0.0, 100.0))
        put(ee_ref, -jnp.sum(un * jnp.log(un + 1e-10)))


def _run(flat, gate_Wt, noise):
    fo = jax.ShapeDtypeStruct
    return pl.pallas_call(
        _router_kernel,
        grid=(_GRID,),
        in_specs=[
            pl.BlockSpec((_TB, _H // 2), lambda i: (i, 0)),
            pl.BlockSpec((_TB, _H // 2), lambda i: (i, 1)),
            pl.BlockSpec((_H, _E), lambda i: (0, 0)),
            pl.BlockSpec((_TB, _E), lambda i: (i, 0)),
        ],
        out_specs=[
            pl.BlockSpec((_TB, _E * _K), lambda i: (i, 0)),
            pl.BlockSpec((_TB, _E * _K), lambda i: (i, 0)),
            pl.BlockSpec((1, _E), lambda i: (0, 0)),
            pl.BlockSpec((1, 1), lambda i: (0, 0)),
            pl.BlockSpec((1, 1), lambda i: (0, 0)),
            pl.BlockSpec((1, 1), lambda i: (0, 0)),
            pl.BlockSpec((1, 1), lambda i: (0, 0)),
            pl.BlockSpec((1, 1), lambda i: (0, 0)),
        ],
        out_shape=[
            fo((_T, _E * _K), jnp.float32),
            fo((_T, _E * _K), jnp.float32),
            fo((1, _E), jnp.float32),
            fo((1, 1), jnp.float32),
            fo((1, 1), jnp.float32),
            fo((1, 1), jnp.float32),
            fo((1, 1), jnp.float32),
            fo((1, 1), jnp.float32),
        ],
        scratch_shapes=[
            pltpu.VMEM((1, _E), jnp.float32),
            pltpu.VMEM((1, _E), jnp.float32),
            pltpu.VMEM((1, 1), jnp.float32),
            pltpu.VMEM((1, 1), jnp.float32),
        ],
    )(flat, flat, gate_Wt, noise)


def kernel(hidden_states, gate_W):
    b, s, h = hidden_states.shape
    flat = hidden_states.reshape(b * s, h).astype(jnp.float32)

    u = jax.random.uniform(jax.random.key(1), (b * s, _E), dtype=jnp.float32)
    noise = -jnp.log(-jnp.log(u + 1e-10) + 1e-10) * 0.1

    (disp, comb, un, ll, il, zl, el, ee) = _run(flat, gate_W.T, noise)

    dispatch = disp.astype(bool).reshape(b, s, _E, _K)
    combine = comb.reshape(b, s, _E, _K)
    return (dispatch, combine,
            ll[0, 0], il[0, 0], zl[0, 0], el[0, 0],
            un[0], ee[0, 0])


# speedup vs baseline: 1.2441x; 1.0238x over previous
import jax
import jax.numpy as jnp
from jax.experimental import pallas as pl
from jax.experimental.pallas import tpu as pltpu

_B, _S, _H = 4, 4096, 2048
_E, _K = 16, 2
_T = _B * _S
_TB = 1024
_GRID = _T // (2 * _TB)
_MIN_PROB = 0.001


def _route_block(h, wt, noise):
    raw = jnp.dot(h, wt, preferred_element_type=jnp.float32)
    logits = raw + noise

    m = jnp.max(logits, axis=-1, keepdims=True)
    ex = jnp.exp(logits - m)
    sm = ex / jnp.sum(ex, axis=-1, keepdims=True)

    scores = sm * (1.0 - _MIN_PROB * _E) + _MIN_PROB
    scores = scores / jnp.sum(scores, axis=-1, keepdims=True)

    e_iota = jax.lax.broadcasted_iota(jnp.int32, scores.shape, 1)
    m1 = jnp.max(scores, axis=-1, keepdims=True)
    i1 = jnp.min(jnp.where(scores == m1, e_iota, _E), axis=-1, keepdims=True)
    masked = jnp.where(e_iota == i1, -jnp.inf, scores)
    m2 = jnp.max(masked, axis=-1, keepdims=True)
    i2 = jnp.min(jnp.where(masked == m2, e_iota, _E), axis=-1, keepdims=True)

    denom = m1 + m2
    w1 = m1 / denom
    w2 = m2 / denom

    lane = jax.lax.broadcasted_iota(jnp.int32, (_TB, _E * _K), 1)
    e_idx = lane // _K
    is0 = (lane - e_idx * _K) == 0
    hit = e_idx == jnp.where(is0, i1, i2)
    comb = jnp.where(hit, jnp.where(is0, w1, w2), 0.0)

    d01 = (e_iota == i1).astype(jnp.float32) + (e_iota == i2).astype(jnp.float32)
    ssum = jnp.sum(scores, axis=0, keepdims=True)
    usum = jnp.sum(d01, axis=0, keepdims=True)
    ent = -jnp.sum(scores * jnp.log(scores + 1e-10), axis=-1, keepdims=True)
    esum = jnp.sum(ent, axis=0, keepdims=True)
    ms = jnp.max(scores, axis=-1, keepdims=True)
    lse = ms + jnp.log(jnp.sum(jnp.exp(scores - ms), axis=-1, keepdims=True))
    zsum = jnp.sum(lse * lse, axis=0, keepdims=True)
    return hit, comb, ssum, usum, esum, zsum


def _router_kernel(ha_ref, hb_ref, wt_ref, noise_ref,
                   disp_ref, comb_ref,
                   un_ref, ll_ref, il_ref, zl_ref, el_ref, ee_ref,
                   acc_scores, acc_usage, acc_ent, acc_z):
    i = pl.program_id(0)

    @pl.when(i == 0)
    def _init():
        acc_scores[...] = jnp.zeros_like(acc_scores)
        acc_usage[...] = jnp.zeros_like(acc_usage)
        acc_ent[...] = jnp.zeros_like(acc_ent)
        acc_z[...] = jnp.zeros_like(acc_z)

    wt = wt_ref[...]
    da, ca, sa, ua, ea, za = _route_block(
        ha_ref[...], wt, noise_ref[:_TB])
    db, cb, sb, ub, eb, zb = _route_block(
        hb_ref[...], wt, noise_ref[_TB:])
    disp_ref[:_TB] = da
    disp_ref[_TB:] = db
    comb_ref[:_TB] = ca
    comb_ref[_TB:] = cb

    acc_scores[...] += sa + sb
    acc_usage[...] += ua + ub
    acc_ent[...] += ea + eb
    acc_z[...] += za + zb

    @pl.when(i == _GRID - 1)
    def _finalize():
        def put(ref, val):
            ref[...] = jnp.broadcast_to(val, (1, 1)).astype(jnp.float32)

        usage = acc_usage[...]
        total = jnp.sum(usage) + 1e-10
        un = usage / total
        un_ref[...] = un
        gate_probs = acc_scores[...] / _T
        put(ll_ref, jnp.clip(jnp.sum(gate_probs * un) * _E, 0.0, 10.0))
        imp_mean = jnp.mean(gate_probs)
        imp_var = jnp.mean((gate_probs - imp_mean) ** 2)
        put(il_ref, jnp.clip(imp_var / (imp_mean * imp_mean + 1e-10), 0.0, 1.0))
        ent_mean = acc_ent[...] / _T
        put(el_ref, jnp.clip((1.0 - ent_mean / jnp.log(jnp.float32(_E))) * 5.0,
                             0.0, 5.0))
        put(zl_ref, jnp.clip(acc_z[...] / _T, 0.0, 100.0))
        put(ee_ref, -jnp.sum(un * jnp.log(un + 1e-10)))


def _run(flat, gate_Wt, noise):
    fo = jax.ShapeDtypeStruct
    return pl.pallas_call(
        _router_kernel,
        grid=(_GRID,),
        in_specs=[
            pl.BlockSpec((_TB, _H), lambda i: (2 * i, 0)),
            pl.BlockSpec((_TB, _H), lambda i: (2 * i + 1, 0)),
            pl.BlockSpec((_H, _E), lambda i: (0, 0)),
            pl.BlockSpec((2 * _TB, _E), lambda i: (i, 0)),
        ],
        out_specs=[
            pl.BlockSpec((2 * _TB, _E * _K), lambda i: (i, 0)),
            pl.BlockSpec((2 * _TB, _E * _K), lambda i: (i, 0)),
            pl.BlockSpec((1, _E), lambda i: (0, 0)),
            pl.BlockSpec((1, 1), lambda i: (0, 0)),
            pl.BlockSpec((1, 1), lambda i: (0, 0)),
            pl.BlockSpec((1, 1), lambda i: (0, 0)),
            pl.BlockSpec((1, 1), lambda i: (0, 0)),
            pl.BlockSpec((1, 1), lambda i: (0, 0)),
        ],
        out_shape=[
            fo((_T, _E * _K), jnp.bool_),
            fo((_T, _E * _K), jnp.float32),
            fo((1, _E), jnp.float32),
            fo((1, 1), jnp.float32),
            fo((1, 1), jnp.float32),
            fo((1, 1), jnp.float32),
            fo((1, 1), jnp.float32),
            fo((1, 1), jnp.float32),
        ],
        scratch_shapes=[
            pltpu.VMEM((1, _E), jnp.float32),
            pltpu.VMEM((1, _E), jnp.float32),
            pltpu.VMEM((1, 1), jnp.float32),
            pltpu.VMEM((1, 1), jnp.float32),
        ],
    )(flat, flat, gate_Wt, noise)


def kernel(hidden_states, gate_W):
    b, s, h = hidden_states.shape
    flat = hidden_states.reshape(b * s, h).astype(jnp.float32)

    u = jax.random.uniform(jax.random.key(1), (b * s, _E), dtype=jnp.float32)
    noise = -jnp.log(-jnp.log(u + 1e-10) + 1e-10) * 0.1

    (disp, comb, un, ll, il, zl, el, ee) = _run(flat, gate_W.T, noise)

    dispatch = disp.reshape(b, s, _E, _K)
    combine = comb.reshape(b, s, _E, _K)
    return (dispatch, combine,
            ll[0, 0], il[0, 0], zl[0, 0], el[0, 0],
            un[0], ee[0, 0])


# speedup vs baseline: 1.2684x; 1.0195x over previous
import jax
import jax.numpy as jnp
from jax.experimental import pallas as pl
from jax.experimental.pallas import tpu as pltpu

_B, _S, _H = 4, 4096, 2048
_E, _K = 16, 2
_T = _B * _S
_TB = 512
_NS = 4
_GRID = _T // (_NS * _TB)
_MIN_PROB = 0.001


def _route_block(h, wt, noise):
    raw = jnp.dot(h, wt, preferred_element_type=jnp.float32)
    logits = raw + noise

    m = jnp.max(logits, axis=-1, keepdims=True)
    ex = jnp.exp(logits - m)
    sm = ex / jnp.sum(ex, axis=-1, keepdims=True)

    scores = sm * (1.0 - _MIN_PROB * _E) + _MIN_PROB
    scores = scores / jnp.sum(scores, axis=-1, keepdims=True)

    e_iota = jax.lax.broadcasted_iota(jnp.int32, scores.shape, 1)
    m1 = jnp.max(scores, axis=-1, keepdims=True)
    i1 = jnp.min(jnp.where(scores == m1, e_iota, _E), axis=-1, keepdims=True)
    masked = jnp.where(e_iota == i1, -jnp.inf, scores)
    m2 = jnp.max(masked, axis=-1, keepdims=True)
    i2 = jnp.min(jnp.where(masked == m2, e_iota, _E), axis=-1, keepdims=True)

    denom = m1 + m2
    w1 = m1 / denom
    w2 = m2 / denom

    lane = jax.lax.broadcasted_iota(jnp.int32, (_TB, _E * _K), 1)
    e_idx = lane // _K
    is0 = (lane - e_idx * _K) == 0
    hit = e_idx == jnp.where(is0, i1, i2)
    comb = jnp.where(hit, jnp.where(is0, w1, w2), 0.0)

    d01 = (e_iota == i1).astype(jnp.float32) + (e_iota == i2).astype(jnp.float32)
    ssum = jnp.sum(scores, axis=0, keepdims=True)
    usum = jnp.sum(d01, axis=0, keepdims=True)
    ent = -jnp.sum(scores * jnp.log(scores + 1e-10), axis=-1, keepdims=True)
    esum = jnp.sum(ent, axis=0, keepdims=True)
    ms = jnp.max(scores, axis=-1, keepdims=True)
    lse = ms + jnp.log(jnp.sum(jnp.exp(scores - ms), axis=-1, keepdims=True))
    zsum = jnp.sum(lse * lse, axis=0, keepdims=True)
    return hit, comb, ssum, usum, esum, zsum


def _router_kernel(*refs):
    h_refs = refs[:_NS]
    wt_ref, noise_ref = refs[_NS], refs[_NS + 1]
    disp_ref, comb_ref = refs[_NS + 2], refs[_NS + 3]
    (un_ref, ll_ref, il_ref, zl_ref, el_ref, ee_ref,
     acc_scores, acc_usage, acc_ent, acc_z) = refs[_NS + 4:]
    i = pl.program_id(0)

    @pl.when(i == 0)
    def _init():
        acc_scores[...] = jnp.zeros_like(acc_scores)
        acc_usage[...] = jnp.zeros_like(acc_usage)
        acc_ent[...] = jnp.zeros_like(acc_ent)
        acc_z[...] = jnp.zeros_like(acc_z)

    wt = wt_ref[...]
    ss = jnp.zeros((1, _E), jnp.float32)
    us = jnp.zeros((1, _E), jnp.float32)
    es = jnp.zeros((1, 1), jnp.float32)
    zs = jnp.zeros((1, 1), jnp.float32)
    for j in range(_NS):
        d, c, s_, u_, e_, z_ = _route_block(
            h_refs[j][...], wt, noise_ref[j * _TB:(j + 1) * _TB])
        disp_ref[j * _TB:(j + 1) * _TB] = d
        comb_ref[j * _TB:(j + 1) * _TB] = c
        ss += s_
        us += u_
        es += e_
        zs += z_

    acc_scores[...] += ss
    acc_usage[...] += us
    acc_ent[...] += es
    acc_z[...] += zs

    @pl.when(i == _GRID - 1)
    def _finalize():
        def put(ref, val):
            ref[...] = jnp.broadcast_to(val, (1, 1)).astype(jnp.float32)

        usage = acc_usage[...]
        total = jnp.sum(usage) + 1e-10
        un = usage / total
        un_ref[...] = un
        gate_probs = acc_scores[...] / _T
        put(ll_ref, jnp.clip(jnp.sum(gate_probs * un) * _E, 0.0, 10.0))
        imp_mean = jnp.mean(gate_probs)
        imp_var = jnp.mean((gate_probs - imp_mean) ** 2)
        put(il_ref, jnp.clip(imp_var / (imp_mean * imp_mean + 1e-10), 0.0, 1.0))
        ent_mean = acc_ent[...] / _T
        put(el_ref, jnp.clip((1.0 - ent_mean / jnp.log(jnp.float32(_E))) * 5.0,
                             0.0, 5.0))
        put(zl_ref, jnp.clip(acc_z[...] / _T, 0.0, 100.0))
        put(ee_ref, -jnp.sum(un * jnp.log(un + 1e-10)))


def _make_h_spec(j):
    return pl.BlockSpec((_TB, _H), lambda i, j=j: (_NS * i + j, 0))


def _run(flat, gate_Wt, noise):
    fo = jax.ShapeDtypeStruct
    const = lambda i: (0, 0)
    return pl.pallas_call(
        _router_kernel,
        grid=(_GRID,),
        in_specs=[_make_h_spec(j) for j in range(_NS)] + [
            pl.BlockSpec((_H, _E), const),
            pl.BlockSpec((_NS * _TB, _E), lambda i: (i, 0)),
        ],
        out_specs=[
            pl.BlockSpec((_NS * _TB, _E * _K), lambda i: (i, 0)),
            pl.BlockSpec((_NS * _TB, _E * _K), lambda i: (i, 0)),
            pl.BlockSpec((1, _E), const),
            pl.BlockSpec((1, 1), const),
            pl.BlockSpec((1, 1), const),
            pl.BlockSpec((1, 1), const),
            pl.BlockSpec((1, 1), const),
            pl.BlockSpec((1, 1), const),
        ],
        out_shape=[
            fo((_T, _E * _K), jnp.bool_),
            fo((_T, _E * _K), jnp.float32),
            fo((1, _E), jnp.float32),
            fo((1, 1), jnp.float32),
            fo((1, 1), jnp.float32),
            fo((1, 1), jnp.float32),
            fo((1, 1), jnp.float32),
            fo((1, 1), jnp.float32),
        ],
        scratch_shapes=[
            pltpu.VMEM((1, _E), jnp.float32),
            pltpu.VMEM((1, _E), jnp.float32),
            pltpu.VMEM((1, 1), jnp.float32),
            pltpu.VMEM((1, 1), jnp.float32),
        ],
    )(*([flat] * _NS), gate_Wt, noise)


def kernel(hidden_states, gate_W):
    b, s, h = hidden_states.shape
    flat = hidden_states.reshape(b * s, h).astype(jnp.float32)

    u = jax.random.uniform(jax.random.key(1), (b * s, _E), dtype=jnp.float32)
    noise = -jnp.log(-jnp.log(u + 1e-10) + 1e-10) * 0.1

    (disp, comb, un, ll, il, zl, el, ee) = _run(flat, gate_W.T, noise)

    dispatch = disp.reshape(b, s, _E, _K)
    combine = comb.reshape(b, s, _E, _K)
    return (dispatch, combine,
            ll[0, 0], il[0, 0], zl[0, 0], el[0, 0],
            un[0], ee[0, 0])
